# baseline (speedup 1.0000x reference)
import functools

import jax
import jax.numpy as jnp
from jax import lax
from jax.experimental import pallas as pl
from jax.experimental.pallas import tpu as pltpu
from jax.experimental.pallas import tpu_sc as plsc

S = 8192
N = 4
D = 768
B = S * N
L = 16
DCH = D // L

NC = 2
NS = 16
NW = NC * NS
B_PER_W = B // NW
CHUNK = 32
NBUF = 4
NCHUNK = B_PER_W // CHUNK
NROUND = NCHUNK // NBUF
NHALF = 2
JH = DCH // NHALF
NG = CHUNK // L


@functools.partial(
    pl.kernel,
    mesh=plsc.VectorSubcoreMesh(core_axis_name="c", subcore_axis_name="s"),
    out_type=jax.ShapeDtypeStruct((S, N, D), jnp.float32),
    scratch_types=[
        pltpu.VMEM((B_PER_W,), jnp.int32),
        pltpu.VMEM((NBUF, CHUNK // N, N, D), jnp.float32),
        pltpu.VMEM((2, D), jnp.float32),
        pltpu.SemaphoreType.DMA,
        pltpu.SemaphoreType.DMA,
        pltpu.SemaphoreType.DMA,
        pltpu.SemaphoreType.DMA,
    ],
)
def _build_body(table_hbm, tf_hbm, out_hbm, tf_v, rows_v, tab_v,
                w0, w1, w2, w3):
    wid = lax.axis_index("s") * NC + lax.axis_index("c")
    pltpu.sync_copy(table_hbm, tab_v)
    pltpu.sync_copy(tf_hbm.at[wid], tf_v)
    base = wid * B_PER_W
    wsem = (w0, w1, w2, w3)

    def wait_write(q):
        pltpu.make_async_copy(
            rows_v.at[q], out_hbm.at[pl.ds(0, CHUNK // N)], wsem[q]).wait()

    def round_body(cp, carry):
        for q in range(NBUF):
            ch = cp * NBUF + q
            tok0 = ch * CHUNK

            @pl.when(cp > 0)
            def _(q=q):
                wait_write(q)

            for h in range(NHALF):
                e0 = [tab_v[0, pl.ds((h * JH + j) * L, L)] for j in range(JH)]
                dl = [tab_v[1, pl.ds((h * JH + j) * L, L)] - e0[j]
                      for j in range(JH)]

                def gbody(g, c2, q=q, h=h, e0=e0, dl=dl, tok0=tok0):
                    tvec = lax.convert_element_type(
                        tf_v[pl.ds(tok0 + g * L, L)], jnp.float32)
                    for k in range(L):
                        tv = lax.broadcast_in_dim(tvec[k], (L,), ())
                        i = g * L + k
                        for j in range(JH):
                            rows_v[q, i // N, i % N,
                                   pl.ds((h * JH + j) * L, L)] = (
                                e0[j] + tv * dl[j])
                    return c2

                lax.fori_loop(0, NG, gbody, 0)

            pltpu.async_copy(
                rows_v.at[q],
                out_hbm.at[pl.ds((base + ch * CHUNK) // N, CHUNK // N)],
                wsem[q])
        return carry

    lax.fori_loop(0, NROUND, round_body, 0)
    for q in range(NBUF):
        wait_write(q)


def kernel(seq_input, token_type_input, token_type_embeddings):
    del seq_input
    ti = token_type_input.reshape(NW, B_PER_W)
    return _build_body(token_type_embeddings, ti)


# speedup vs baseline: 1.4365x; 1.4365x over previous
import functools

import jax
import jax.numpy as jnp
from jax import lax
from jax.experimental import pallas as pl
from jax.experimental.pallas import tpu as pltpu
from jax.experimental.pallas import tpu_sc as plsc

S = 8192
N = 4
D = 768
B = S * N
L = 16
DCH = D // L

NC = 2
NS = 16
NW = NC * NS
B_PER_W = B // NW
CHUNK = 64
NCHUNK = B_PER_W // CHUNK
NPAIR = NCHUNK // 2
NHALF = 2
JH = DCH // NHALF
NG = CHUNK // L


@functools.partial(
    pl.kernel,
    mesh=plsc.VectorSubcoreMesh(core_axis_name="c", subcore_axis_name="s"),
    out_type=jax.ShapeDtypeStruct((S, N, D), jnp.float32),
    scratch_types=[
        pltpu.VMEM((B_PER_W,), jnp.int32),
        pltpu.VMEM((2, CHUNK // N, N, D), jnp.float32),
        pltpu.VMEM((2, D), jnp.float32),
        pltpu.VMEM_SHARED((2, D), jnp.float32),
        pltpu.SemaphoreType.DMA,
        pltpu.SemaphoreType.DMA,
    ],
)
def _build_body(table_hbm, tf_hbm, out_hbm, tf_v, rows_v, tab_v, tab_sh,
                w0, w1):
    wid = lax.axis_index("s") * NC + lax.axis_index("c")

    @pl.when(lax.axis_index("s") == 0)
    def _():
        pltpu.sync_copy(table_hbm, tab_sh)
    pltpu.sync_copy(tf_hbm.at[wid], tf_v)
    plsc.subcore_barrier()
    pltpu.sync_copy(tab_sh, tab_v)
    base = wid * B_PER_W
    wsem = (w0, w1)

    def wait_write(q):
        pltpu.make_async_copy(
            rows_v.at[q], out_hbm.at[pl.ds(0, CHUNK // N)], wsem[q]).wait()

    def pair_body(cp, carry):
        for q in range(2):
            ch = cp * 2 + q
            tok0 = ch * CHUNK

            @pl.when(cp > 0)
            def _(q=q):
                wait_write(q)

            for h in range(NHALF):
                e0 = [tab_v[0, pl.ds((h * JH + j) * L, L)] for j in range(JH)]
                dl = [tab_v[1, pl.ds((h * JH + j) * L, L)] - e0[j]
                      for j in range(JH)]

                def gbody(g, c2, q=q, h=h, e0=e0, dl=dl, tok0=tok0):
                    tvec = lax.convert_element_type(
                        tf_v[pl.ds(tok0 + g * L, L)], jnp.float32)
                    for k in range(L):
                        tv = lax.broadcast_in_dim(tvec[k], (L,), ())
                        i = g * L + k
                        for j in range(JH):
                            rows_v[q, i // N, i % N,
                                   pl.ds((h * JH + j) * L, L)] = (
                                e0[j] + tv * dl[j])
                    return c2

                lax.fori_loop(0, NG, gbody, 0)

            pltpu.async_copy(
                rows_v.at[q],
                out_hbm.at[pl.ds((base + ch * CHUNK) // N, CHUNK // N)],
                wsem[q])
        return carry

    lax.fori_loop(0, NPAIR, pair_body, 0)
    wait_write(0)
    wait_write(1)


def kernel(seq_input, token_type_input, token_type_embeddings):
    del seq_input
    ti = token_type_input.reshape(NW, B_PER_W)
    return _build_body(token_type_embeddings, ti)
